# baseline (speedup 1.0000x reference)
import jax
import jax.numpy as jnp
from jax.experimental import pallas as pl
from jax.experimental.pallas import tpu as pltpu

OBS_DIM = 64
ACT_DIM = 16
HID = 256
LAT = 32
K = 8192
NCHUNK = 2
CHUNK = K // NCHUNK
COMMITMENT_COST = 0.25

T = 256


def _enc_body(x_ref, w1_ref, b1_ref, w2_ref, b2_ref, w3_ref, b3_ref, lat_ref):
    h = jnp.maximum(
        jnp.dot(x_ref[...], w1_ref[...], preferred_element_type=jnp.float32)
        + b1_ref[...], 0.0)
    h = jnp.maximum(
        jnp.dot(h, w2_ref[...], preferred_element_type=jnp.float32)
        + b2_ref[...], 0.0)
    lat_ref[...] = (jnp.dot(h, w3_ref[...], preferred_element_type=jnp.float32)
                    + b3_ref[...])


def _vq_dec_body(lat_ref, fn_ref, cn_ref, cbT_ref, cb0_ref, cb1_ref, cb2_ref,
                 obs_ref, act_ref,
                 dw1_ref, db1_ref, dw2_ref, db2_ref, dw3_ref, db3_ref,
                 recon_ref, tok_ref, qst_ref, sq_ref, rec_ref):
    i = pl.program_id(0)
    lat = lat_ref[...]

    mm = jnp.dot(lat, cbT_ref[...], preferred_element_type=jnp.float32)
    d2 = (fn_ref[...] + cn_ref[...]) - 2.0 * mm

    iota = jax.lax.broadcasted_iota(jnp.int32, (T, CHUNK), 1)
    acc_v = None
    acc_i = None
    for c in range(NCHUNK):
        d2c = d2[:, c * CHUNK:(c + 1) * CHUNK]
        mnc = jnp.min(d2c, axis=1, keepdims=True)
        idxc = jnp.min(jnp.where(d2c == mnc, iota + c * CHUNK, K),
                       axis=1, keepdims=True)
        bits = jax.lax.bitcast_convert_type(mnc, jnp.int32)
        rnd = bits + (jnp.int32(0x7FFF) + ((bits >> 16) & 1))
        qv = jax.lax.bitcast_convert_type(rnd & jnp.int32(-65536), jnp.float32)
        if acc_v is None:
            acc_v, acc_i = qv, idxc
        else:
            win = mnc < acc_v
            acc_v = jnp.where(win, qv, acc_v)
            acc_i = jnp.where(win, idxc, acc_i)
    tok_ref[...] = acc_i

    iota_k = jax.lax.broadcasted_iota(jnp.int32, (T, K), 1)
    oh = (iota_k == acc_i).astype(jnp.float32)
    dn = (((1,), (0,)), ((), ()))
    quant = (jax.lax.dot_general(oh, cb0_ref[...], dn,
                                 preferred_element_type=jnp.float32)
             + jax.lax.dot_general(oh, cb1_ref[...], dn,
                                   preferred_element_type=jnp.float32)
             + jax.lax.dot_general(oh, cb2_ref[...], dn,
                                   preferred_element_type=jnp.float32))

    qst = lat + (quant - lat)
    qst_ref[...] = qst
    diff = lat - quant

    y = jnp.concatenate([qst, act_ref[...]], axis=1)
    hd = jnp.maximum(
        jnp.dot(y, dw1_ref[...], preferred_element_type=jnp.float32)
        + db1_ref[...], 0.0)
    hd = jnp.maximum(
        jnp.dot(hd, dw2_ref[...], preferred_element_type=jnp.float32)
        + db2_ref[...], 0.0)
    recon = (jnp.dot(hd, dw3_ref[...], preferred_element_type=jnp.float32)
             + db3_ref[...])
    recon_ref[...] = recon

    rd = recon - obs_ref[...]
    sq = jnp.sum(diff * diff).reshape(1, 1)
    rc = jnp.sum(rd * rd).reshape(1, 1)

    @pl.when(i == 0)
    def _init():
        sq_ref[...] = jnp.zeros_like(sq_ref)
        rec_ref[...] = jnp.zeros_like(rec_ref)

    sq_ref[...] += sq
    rec_ref[...] += rc


def kernel(obs, actions, enc_W1, enc_b1, enc_W2, enc_b2, enc_W3, enc_b3,
           codebook, dec_W1, dec_b1, dec_W2, dec_b2, dec_W3, dec_b3):
    b, s, _ = obs.shape
    n = b * s
    nb = n // T

    x = jnp.concatenate([obs, actions], axis=-1).reshape(n, OBS_DIM + ACT_DIM)
    obs2 = obs.reshape(n, OBS_DIM)
    act2 = actions.reshape(n, ACT_DIM)
    cbT = codebook.T
    cb0 = codebook.astype(jnp.bfloat16).astype(jnp.float32)
    r1 = codebook - cb0
    cb1 = r1.astype(jnp.bfloat16).astype(jnp.float32)
    cb2 = (r1 - cb1).astype(jnp.bfloat16).astype(jnp.float32)

    full = lambda shp: pl.BlockSpec(shp, lambda i: (0,) * len(shp))
    row = lambda w: pl.BlockSpec((T, w), lambda i: (i, 0))

    lat = pl.pallas_call(
        _enc_body,
        grid=(nb,),
        in_specs=[row(OBS_DIM + ACT_DIM),
                  full((OBS_DIM + ACT_DIM, HID)), full((1, HID)),
                  full((HID, HID)), full((1, HID)),
                  full((HID, LAT)), full((1, LAT))],
        out_specs=row(LAT),
        out_shape=jax.ShapeDtypeStruct((n, LAT), jnp.float32),
        compiler_params=pltpu.CompilerParams(
            dimension_semantics=("arbitrary",)),
    )(x, enc_W1, enc_b1.reshape(1, HID), enc_W2, enc_b2.reshape(1, HID),
      enc_W3, enc_b3.reshape(1, LAT))

    fn = jnp.sum(lat ** 2, axis=1, keepdims=True)
    cn = jnp.sum(codebook ** 2, axis=1)[None, :]

    out_shapes = (
        jax.ShapeDtypeStruct((n, OBS_DIM), jnp.float32),
        jax.ShapeDtypeStruct((n, 1), jnp.int32),
        jax.ShapeDtypeStruct((n, LAT), jnp.float32),
        jax.ShapeDtypeStruct((1, 1), jnp.float32),
        jax.ShapeDtypeStruct((1, 1), jnp.float32),
    )
    out_specs = (row(OBS_DIM), row(1), row(LAT), full((1, 1)), full((1, 1)))
    in_specs = [
        row(LAT), row(1), full((1, K)), full((LAT, K)),
        full((K, LAT)), full((K, LAT)), full((K, LAT)),
        row(OBS_DIM), row(ACT_DIM),
        full((LAT + ACT_DIM, HID)), full((1, HID)),
        full((HID, HID)), full((1, HID)),
        full((HID, OBS_DIM)), full((1, OBS_DIM)),
    ]

    recon, tok, qst, sq, rec = pl.pallas_call(
        _vq_dec_body,
        grid=(nb,),
        in_specs=in_specs,
        out_specs=out_specs,
        out_shape=out_shapes,
        compiler_params=pltpu.CompilerParams(
            dimension_semantics=("arbitrary",)),
    )(lat, fn, cn, cbT, cb0, cb1, cb2, obs2, act2,
      dec_W1, dec_b1.reshape(1, HID), dec_W2, dec_b2.reshape(1, HID),
      dec_W3, dec_b3.reshape(1, OBS_DIM))

    m = sq[0, 0] / jnp.float32(n * LAT)
    commitment_loss = m * jnp.float32(COMMITMENT_COST)
    codebook_loss = m
    total_q = commitment_loss + codebook_loss
    recon_loss = rec[0, 0] / jnp.float32(n * OBS_DIM)
    total_loss = recon_loss + total_q

    return (recon.reshape(b, s, OBS_DIM), tok.reshape(b, s),
            qst.reshape(b, s, LAT), lat.reshape(b, s, LAT),
            recon_loss, commitment_loss, codebook_loss, total_loss)


# speedup vs baseline: 1.7458x; 1.7458x over previous
import functools

import jax
import jax.numpy as jnp
from jax import lax
from jax.experimental import pallas as pl
from jax.experimental.pallas import tpu as pltpu
from jax.experimental.pallas import tpu_sc as plsc

OBS_DIM = 64
ACT_DIM = 16
HID = 256
LAT = 32
K = 8192
NCHUNK = 2
CHUNK = K // NCHUNK
COMMITMENT_COST = 0.25

T = 256


def _enc_body(x_ref, w1_ref, b1_ref, w2_ref, b2_ref, w3_ref, b3_ref, lat_ref):
    h = jnp.maximum(
        jnp.dot(x_ref[...], w1_ref[...], preferred_element_type=jnp.float32)
        + b1_ref[...], 0.0)
    h = jnp.maximum(
        jnp.dot(h, w2_ref[...], preferred_element_type=jnp.float32)
        + b2_ref[...], 0.0)
    lat_ref[...] = (jnp.dot(h, w3_ref[...], preferred_element_type=jnp.float32)
                    + b3_ref[...])


def _vq_body(lat_ref, fn_ref, cn_ref, cbT_ref, tok_ref):
    lat = lat_ref[...]
    mm = jnp.dot(lat, cbT_ref[...], preferred_element_type=jnp.float32)
    d2 = (fn_ref[...] + cn_ref[...]) - 2.0 * mm

    iota = jax.lax.broadcasted_iota(jnp.int32, (T, CHUNK), 1)
    acc_v = None
    acc_i = None
    for c in range(NCHUNK):
        d2c = d2[:, c * CHUNK:(c + 1) * CHUNK]
        mnc = jnp.min(d2c, axis=1, keepdims=True)
        idxc = jnp.min(jnp.where(d2c == mnc, iota + c * CHUNK, K),
                       axis=1, keepdims=True)
        bits = jax.lax.bitcast_convert_type(mnc, jnp.int32)
        rnd = bits + (jnp.int32(0x7FFF) + ((bits >> 16) & 1))
        qv = jax.lax.bitcast_convert_type(rnd & jnp.int32(-65536), jnp.float32)
        if acc_v is None:
            acc_v, acc_i = qv, idxc
        else:
            win = mnc < acc_v
            acc_v = jnp.where(win, qv, acc_v)
            acc_i = jnp.where(win, idxc, acc_i)
    tok_ref[...] = acc_i


def _dec_body(lat_ref, quant_ref, obs_ref, act_ref,
              dw1_ref, db1_ref, dw2_ref, db2_ref, dw3_ref, db3_ref,
              recon_ref, qst_ref, sq_ref, rec_ref):
    i = pl.program_id(0)
    lat = lat_ref[...]
    quant = quant_ref[:, :LAT]
    qst = lat + (quant - lat)
    qst_ref[...] = qst
    diff = lat - quant

    y = jnp.concatenate([qst, act_ref[...]], axis=1)
    hd = jnp.maximum(
        jnp.dot(y, dw1_ref[...], preferred_element_type=jnp.float32)
        + db1_ref[...], 0.0)
    hd = jnp.maximum(
        jnp.dot(hd, dw2_ref[...], preferred_element_type=jnp.float32)
        + db2_ref[...], 0.0)
    recon = (jnp.dot(hd, dw3_ref[...], preferred_element_type=jnp.float32)
             + db3_ref[...])
    recon_ref[...] = recon

    rd = recon - obs_ref[...]
    sq = jnp.sum(diff * diff).reshape(1, 1)
    rc = jnp.sum(rd * rd).reshape(1, 1)

    @pl.when(i == 0)
    def _init():
        sq_ref[...] = jnp.zeros_like(sq_ref)
        rec_ref[...] = jnp.zeros_like(rec_ref)

    sq_ref[...] += sq
    rec_ref[...] += rc


GATHER_D = 128


def _make_sc_gather(n):
    info = plsc.get_sparse_core_info()
    nc, ns = info.num_cores, info.num_subcores
    nw = nc * ns
    b_per_w = n // nw
    mesh = plsc.VectorSubcoreMesh(core_axis_name="c", subcore_axis_name="s")

    @functools.partial(
        pl.kernel, mesh=mesh,
        out_type=jax.ShapeDtypeStruct((n, GATHER_D), jnp.float32),
        scratch_types=[
            pltpu.VMEM((b_per_w,), jnp.int32),
            pltpu.VMEM((b_per_w, GATHER_D), jnp.float32),
            pltpu.SemaphoreType.DMA,
        ],
    )
    def gather_k(table_hbm, idx_hbm, out_hbm, idx_v, rows_v, sem):
        wid = lax.axis_index("s") * nc + lax.axis_index("c")
        base = wid * b_per_w
        pltpu.sync_copy(idx_hbm.at[pl.ds(base, b_per_w)], idx_v)
        pltpu.async_copy(table_hbm.at[idx_v], rows_v, sem).wait()
        pltpu.sync_copy(rows_v, out_hbm.at[pl.ds(base, b_per_w)])

    return gather_k


def kernel(obs, actions, enc_W1, enc_b1, enc_W2, enc_b2, enc_W3, enc_b3,
           codebook, dec_W1, dec_b1, dec_W2, dec_b2, dec_W3, dec_b3):
    b, s, _ = obs.shape
    n = b * s
    nb = n // T

    x = jnp.concatenate([obs, actions], axis=-1).reshape(n, OBS_DIM + ACT_DIM)
    obs2 = obs.reshape(n, OBS_DIM)
    act2 = actions.reshape(n, ACT_DIM)
    cbT = codebook.T

    full = lambda shp: pl.BlockSpec(shp, lambda i: (0,) * len(shp))
    row = lambda w: pl.BlockSpec((T, w), lambda i: (i, 0))

    lat = pl.pallas_call(
        _enc_body,
        grid=(nb,),
        in_specs=[row(OBS_DIM + ACT_DIM),
                  full((OBS_DIM + ACT_DIM, HID)), full((1, HID)),
                  full((HID, HID)), full((1, HID)),
                  full((HID, LAT)), full((1, LAT))],
        out_specs=row(LAT),
        out_shape=jax.ShapeDtypeStruct((n, LAT), jnp.float32),
        compiler_params=pltpu.CompilerParams(
            dimension_semantics=("arbitrary",)),
    )(x, enc_W1, enc_b1.reshape(1, HID), enc_W2, enc_b2.reshape(1, HID),
      enc_W3, enc_b3.reshape(1, LAT))

    fn = jnp.sum(lat ** 2, axis=1, keepdims=True)
    cn = jnp.sum(codebook ** 2, axis=1)[None, :]

    tok = pl.pallas_call(
        _vq_body,
        grid=(nb,),
        in_specs=[row(LAT), row(1), full((1, K)), full((LAT, K))],
        out_specs=row(1),
        out_shape=jax.ShapeDtypeStruct((n, 1), jnp.int32),
        compiler_params=pltpu.CompilerParams(
            dimension_semantics=("arbitrary",)),
    )(lat, fn, cn, cbT)

    cb_pad = jnp.pad(codebook, ((0, 0), (0, GATHER_D - LAT)))
    quant_pad = _make_sc_gather(n)(cb_pad, tok.reshape(n))

    out_shapes = (
        jax.ShapeDtypeStruct((n, OBS_DIM), jnp.float32),
        jax.ShapeDtypeStruct((n, LAT), jnp.float32),
        jax.ShapeDtypeStruct((1, 1), jnp.float32),
        jax.ShapeDtypeStruct((1, 1), jnp.float32),
    )
    recon, qst, sq, rec = pl.pallas_call(
        _dec_body,
        grid=(nb,),
        in_specs=[row(LAT), row(GATHER_D), row(OBS_DIM), row(ACT_DIM),
                  full((LAT + ACT_DIM, HID)), full((1, HID)),
                  full((HID, HID)), full((1, HID)),
                  full((HID, OBS_DIM)), full((1, OBS_DIM))],
        out_specs=(row(OBS_DIM), row(LAT), full((1, 1)), full((1, 1))),
        out_shape=out_shapes,
        compiler_params=pltpu.CompilerParams(
            dimension_semantics=("arbitrary",)),
    )(lat, quant_pad, obs2, act2,
      dec_W1, dec_b1.reshape(1, HID), dec_W2, dec_b2.reshape(1, HID),
      dec_W3, dec_b3.reshape(1, OBS_DIM))

    m = sq[0, 0] / jnp.float32(n * LAT)
    commitment_loss = m * jnp.float32(COMMITMENT_COST)
    codebook_loss = m
    total_q = commitment_loss + codebook_loss
    recon_loss = rec[0, 0] / jnp.float32(n * OBS_DIM)
    total_loss = recon_loss + total_q

    return (recon.reshape(b, s, OBS_DIM), tok.reshape(b, s),
            qst.reshape(b, s, LAT), lat.reshape(b, s, LAT),
            recon_loss, commitment_loss, codebook_loss, total_loss)


# speedup vs baseline: 1.7808x; 1.0200x over previous
import functools

import jax
import jax.numpy as jnp
from jax import lax
from jax.experimental import pallas as pl
from jax.experimental.pallas import tpu as pltpu
from jax.experimental.pallas import tpu_sc as plsc

OBS_DIM = 64
ACT_DIM = 16
HID = 256
LAT = 32
K = 8192
NCHUNK = 2
CHUNK = K // NCHUNK
COMMITMENT_COST = 0.25

T = 256
TV = 512


def _enc_body(x_ref, w1_ref, b1_ref, w2_ref, b2_ref, w3_ref, b3_ref, lat_ref):
    h = jnp.maximum(
        jnp.dot(x_ref[...], w1_ref[...], preferred_element_type=jnp.float32)
        + b1_ref[...], 0.0)
    h = jnp.maximum(
        jnp.dot(h, w2_ref[...], preferred_element_type=jnp.float32)
        + b2_ref[...], 0.0)
    lat_ref[...] = (jnp.dot(h, w3_ref[...], preferred_element_type=jnp.float32)
                    + b3_ref[...])


def _vq_body(lat_ref, fn_ref, cn_ref, cbT_ref, tok_ref):
    lat = lat_ref[...]
    mm = jnp.dot(lat, cbT_ref[...], preferred_element_type=jnp.float32)
    d2 = (fn_ref[...] + cn_ref[...]) - 2.0 * mm

    iota = jax.lax.broadcasted_iota(jnp.int32, (TV, CHUNK), 1)
    acc_v = None
    acc_i = None
    for c in range(NCHUNK):
        d2c = d2[:, c * CHUNK:(c + 1) * CHUNK]
        mnc = jnp.min(d2c, axis=1, keepdims=True)
        idxc = jnp.min(jnp.where(d2c == mnc, iota + c * CHUNK, K),
                       axis=1, keepdims=True)
        bits = jax.lax.bitcast_convert_type(mnc, jnp.int32)
        rnd = bits + (jnp.int32(0x7FFF) + ((bits >> 16) & 1))
        qv = jax.lax.bitcast_convert_type(rnd & jnp.int32(-65536), jnp.float32)
        if acc_v is None:
            acc_v, acc_i = qv, idxc
        else:
            win = mnc < acc_v
            acc_v = jnp.where(win, qv, acc_v)
            acc_i = jnp.where(win, idxc, acc_i)
    tok_ref[...] = acc_i


def _dec_body(lat_ref, quant_ref, obs_ref, act_ref,
              dw1_ref, db1_ref, dw2_ref, db2_ref, dw3_ref, db3_ref,
              recon_ref, qst_ref, sq_ref, rec_ref):
    i = pl.program_id(0)
    lat = lat_ref[...]
    quant = quant_ref[:, :LAT]
    qst = lat + (quant - lat)
    qst_ref[...] = qst
    diff = lat - quant

    y = jnp.concatenate([qst, act_ref[...]], axis=1)
    hd = jnp.maximum(
        jnp.dot(y, dw1_ref[...], preferred_element_type=jnp.float32)
        + db1_ref[...], 0.0)
    hd = jnp.maximum(
        jnp.dot(hd, dw2_ref[...], preferred_element_type=jnp.float32)
        + db2_ref[...], 0.0)
    recon = (jnp.dot(hd, dw3_ref[...], preferred_element_type=jnp.float32)
             + db3_ref[...])
    recon_ref[...] = recon

    rd = recon - obs_ref[...]
    sq = jnp.sum(diff * diff).reshape(1, 1)
    rc = jnp.sum(rd * rd).reshape(1, 1)

    @pl.when(i == 0)
    def _init():
        sq_ref[...] = jnp.zeros_like(sq_ref)
        rec_ref[...] = jnp.zeros_like(rec_ref)

    sq_ref[...] += sq
    rec_ref[...] += rc


GATHER_D = 128


def _make_sc_gather(n):
    info = plsc.get_sparse_core_info()
    nc, ns = info.num_cores, info.num_subcores
    nw = nc * ns
    b_per_w = n // nw
    mesh = plsc.VectorSubcoreMesh(core_axis_name="c", subcore_axis_name="s")

    @functools.partial(
        pl.kernel, mesh=mesh,
        out_type=jax.ShapeDtypeStruct((n, GATHER_D), jnp.float32),
        scratch_types=[
            pltpu.VMEM((b_per_w,), jnp.int32),
            pltpu.VMEM((b_per_w, GATHER_D), jnp.float32),
            pltpu.SemaphoreType.DMA,
        ],
    )
    def gather_k(table_hbm, idx_hbm, out_hbm, idx_v, rows_v, sem):
        wid = lax.axis_index("s") * nc + lax.axis_index("c")
        base = wid * b_per_w
        pltpu.sync_copy(idx_hbm.at[pl.ds(base, b_per_w)], idx_v)
        pltpu.async_copy(table_hbm.at[idx_v], rows_v, sem).wait()
        pltpu.sync_copy(rows_v, out_hbm.at[pl.ds(base, b_per_w)])

    return gather_k


def kernel(obs, actions, enc_W1, enc_b1, enc_W2, enc_b2, enc_W3, enc_b3,
           codebook, dec_W1, dec_b1, dec_W2, dec_b2, dec_W3, dec_b3):
    b, s, _ = obs.shape
    n = b * s
    nb = n // T

    x = jnp.concatenate([obs, actions], axis=-1).reshape(n, OBS_DIM + ACT_DIM)
    obs2 = obs.reshape(n, OBS_DIM)
    act2 = actions.reshape(n, ACT_DIM)
    cbT = codebook.T

    full = lambda shp: pl.BlockSpec(shp, lambda i: (0,) * len(shp))
    row = lambda w: pl.BlockSpec((T, w), lambda i: (i, 0))

    lat = pl.pallas_call(
        _enc_body,
        grid=(nb,),
        in_specs=[row(OBS_DIM + ACT_DIM),
                  full((OBS_DIM + ACT_DIM, HID)), full((1, HID)),
                  full((HID, HID)), full((1, HID)),
                  full((HID, LAT)), full((1, LAT))],
        out_specs=row(LAT),
        out_shape=jax.ShapeDtypeStruct((n, LAT), jnp.float32),
        compiler_params=pltpu.CompilerParams(
            dimension_semantics=("arbitrary",)),
    )(x, enc_W1, enc_b1.reshape(1, HID), enc_W2, enc_b2.reshape(1, HID),
      enc_W3, enc_b3.reshape(1, LAT))

    fn = jnp.sum(lat ** 2, axis=1, keepdims=True)
    cn = jnp.sum(codebook ** 2, axis=1)[None, :]

    rowv = lambda w: pl.BlockSpec((TV, w), lambda i: (i, 0))
    tok = pl.pallas_call(
        _vq_body,
        grid=(n // TV,),
        in_specs=[rowv(LAT), rowv(1), full((1, K)), full((LAT, K))],
        out_specs=rowv(1),
        out_shape=jax.ShapeDtypeStruct((n, 1), jnp.int32),
        compiler_params=pltpu.CompilerParams(
            dimension_semantics=("arbitrary",)),
    )(lat, fn, cn, cbT)

    cb_pad = jnp.pad(codebook, ((0, 0), (0, GATHER_D - LAT)))
    quant_pad = _make_sc_gather(n)(cb_pad, tok.reshape(n))

    out_shapes = (
        jax.ShapeDtypeStruct((n, OBS_DIM), jnp.float32),
        jax.ShapeDtypeStruct((n, LAT), jnp.float32),
        jax.ShapeDtypeStruct((1, 1), jnp.float32),
        jax.ShapeDtypeStruct((1, 1), jnp.float32),
    )
    recon, qst, sq, rec = pl.pallas_call(
        _dec_body,
        grid=(nb,),
        in_specs=[row(LAT), row(GATHER_D), row(OBS_DIM), row(ACT_DIM),
                  full((LAT + ACT_DIM, HID)), full((1, HID)),
                  full((HID, HID)), full((1, HID)),
                  full((HID, OBS_DIM)), full((1, OBS_DIM))],
        out_specs=(row(OBS_DIM), row(LAT), full((1, 1)), full((1, 1))),
        out_shape=out_shapes,
        compiler_params=pltpu.CompilerParams(
            dimension_semantics=("arbitrary",)),
    )(lat, quant_pad, obs2, act2,
      dec_W1, dec_b1.reshape(1, HID), dec_W2, dec_b2.reshape(1, HID),
      dec_W3, dec_b3.reshape(1, OBS_DIM))

    m = sq[0, 0] / jnp.float32(n * LAT)
    commitment_loss = m * jnp.float32(COMMITMENT_COST)
    codebook_loss = m
    total_q = commitment_loss + codebook_loss
    recon_loss = rec[0, 0] / jnp.float32(n * OBS_DIM)
    total_loss = recon_loss + total_q

    return (recon.reshape(b, s, OBS_DIM), tok.reshape(b, s),
            qst.reshape(b, s, LAT), lat.reshape(b, s, LAT),
            recon_loss, commitment_loss, codebook_loss, total_loss)
